# baseline (speedup 1.0000x reference)
import functools

import jax
import jax.numpy as jnp
from jax import lax
from jax.experimental import pallas as pl
from jax.experimental.pallas import tpu as pltpu
from jax.experimental.pallas import tpu_sc as plsc

NC = 2
NS = 16
LANES = 16

CHUNK = 32
RING = 9
GLAG = 7


def _sc_partial_agg(x, edge_index, n_pad):
    n, d = x.shape
    e = edge_index.shape[1]
    e_sub = e // (NC * NS)
    n_chunks = e_sub // CHUNK
    tail = e_sub - n_chunks * CHUNK
    rows_per_sub = n_pad // NS
    dummy = n

    zeros = jnp.zeros((rows_per_sub, d), jnp.float32)

    mesh = plsc.VectorSubcoreMesh(core_axis_name="c", subcore_axis_name="s")

    @functools.partial(
        pl.kernel,
        out_type=jax.ShapeDtypeStruct((NC, n_pad, d), jnp.float32),
        mesh=mesh,
        scratch_types=[
            pltpu.VMEM((RING, CHUNK), jnp.int32),
            pltpu.VMEM((RING, CHUNK), jnp.int32),
            pltpu.VMEM((RING, CHUNK), jnp.int32),
            pltpu.VMEM((RING, CHUNK, d), jnp.float32),
            pltpu.VMEM_SHARED((n_pad, d), jnp.float32),
            pltpu.SemaphoreType.DMA((RING,)),
            pltpu.SemaphoreType.DMA((RING,)),
            pltpu.SemaphoreType.DMA((RING,)),
            pltpu.SemaphoreType.DMA,
        ],
    )
    def sc_kernel(x_hbm, ei_hbm, z_hbm, out_hbm,
                  dstb, srcb, dstm, rows, acc, isem, gsem, ssem, zsem):
        c = lax.axis_index("c")
        s = lax.axis_index("s")
        w = c * NS + s
        base = w * e_sub

        def start_idx(k, b):
            pltpu.async_copy(
                ei_hbm.at[pl.ds(base + k * CHUNK, CHUNK)], dstb.at[b],
                isem.at[b])
            pltpu.async_copy(
                ei_hbm.at[pl.ds(e + base + k * CHUNK, CHUNK)], srcb.at[b],
                isem.at[b])

        def wait_idx(b):
            pltpu.make_async_copy(
                ei_hbm.at[pl.ds(0, CHUNK)], dstb.at[b], isem.at[b]).wait()
            pltpu.make_async_copy(
                ei_hbm.at[pl.ds(0, CHUNK)], srcb.at[b], isem.at[b]).wait()

        def start_gather(b):
            pltpu.async_copy(
                x_hbm.at[srcb.at[b]], rows.at[b], gsem.at[b])

        def wait_gather(b):
            pltpu.make_async_copy(
                x_hbm.at[srcb.at[b]], rows.at[b], gsem.at[b]).wait()

        def start_scatter(b):
            pltpu.async_copy(
                rows.at[b], acc.at[dstm.at[b]], ssem.at[b], add=True)

        def wait_scatter(b):
            pltpu.make_async_copy(
                rows.at[b], acc.at[dstm.at[0]], ssem.at[b]).wait()

        def mask(b):
            for i in range(0, CHUNK, LANES):
                dsl = dstb[b, pl.ds(i, LANES)]
                ssl = srcb[b, pl.ds(i, LANES)]
                dstm[b, pl.ds(i, LANES)] = jnp.where(dsl != ssl, dsl, dummy)

        q = (rows_per_sub // 32) * 8
        zq = [q, q, q, rows_per_sub - 3 * q]
        zoff = [0, q, 2 * q, 3 * q]
        for j in range(4):
            pltpu.async_copy(
                z_hbm.at[pl.ds(zoff[j], zq[j])],
                acc.at[pl.ds(s * rows_per_sub + zoff[j], zq[j])], zsem)
        for j in range(2):
            start_idx(j, j)
        for j in range(4):
            pltpu.make_async_copy(
                z_hbm.at[pl.ds(zoff[j], zq[j])],
                acc.at[pl.ds(s * rows_per_sub + zoff[j], zq[j])], zsem).wait()
        plsc.subcore_barrier()

        total_steps = n_chunks + RING
        loop_hi = ((total_steps + RING - 1) // RING) * RING

        @pl.loop(0, loop_hi, step=RING)
        def _(k0):
            for j in range(RING):
                k = k0 + j
                b = j
                bg = (j - GLAG) % RING
                bi = (j + 2) % RING

                @pl.when(jnp.logical_and(k >= RING, k < n_chunks + RING))
                def _():
                    wait_scatter(b)

                @pl.when(k < n_chunks)
                def _():
                    wait_idx(b)
                    start_gather(b)
                    mask(b)

                @pl.when(jnp.logical_and(k >= GLAG, k < n_chunks + GLAG))
                def _():
                    wait_gather(bg)
                    start_scatter(bg)

                @pl.when(k + 2 < n_chunks)
                def _():
                    start_idx(k + 2, bi)

        if tail > 0:
            pltpu.sync_copy(
                ei_hbm.at[pl.ds(base + n_chunks * CHUNK, tail)],
                dstb.at[0, pl.ds(0, tail)])
            pltpu.sync_copy(
                ei_hbm.at[pl.ds(e + base + n_chunks * CHUNK, tail)],
                srcb.at[0, pl.ds(0, tail)])
            for i in range(0, CHUNK, LANES):
                if i + LANES <= tail:
                    dsl = dstb[0, pl.ds(i, LANES)]
                    ssl = srcb[0, pl.ds(i, LANES)]
                    dstm[0, pl.ds(i, LANES)] = jnp.where(dsl != ssl, dsl, dummy)
                else:
                    dstm[0, pl.ds(i, LANES)] = jnp.full((LANES,), dummy, jnp.int32)
                    srcb[0, pl.ds(i, LANES)] = jnp.zeros((LANES,), jnp.int32)
            start_gather(0)
            wait_gather(0)
            start_scatter(0)
            wait_scatter(0)

        plsc.subcore_barrier()

        for j in range(4):
            pltpu.async_copy(
                acc.at[pl.ds(s * rows_per_sub + zoff[j], zq[j])],
                out_hbm.at[c, pl.ds(s * rows_per_sub + zoff[j], zq[j])], zsem)
        for j in range(4):
            pltpu.make_async_copy(
                acc.at[pl.ds(s * rows_per_sub + zoff[j], zq[j])],
                out_hbm.at[c, pl.ds(s * rows_per_sub + zoff[j], zq[j])], zsem).wait()

    return sc_kernel(x, edge_index.reshape(-1), zeros)


def _tc_combine_body(eps_ref, x_ref, p_ref, o_ref):
    scale = 1.0 + eps_ref[0]
    o_ref[...] = scale * x_ref[...] + p_ref[0] + p_ref[1]


def kernel(x, edge_index, eps):
    n, d = x.shape
    n_pad = 10112
    partial = _sc_partial_agg(x, edge_index, n_pad)

    blk = 1000
    grid = (n // blk,)
    out = pl.pallas_call(
        _tc_combine_body,
        grid=grid,
        in_specs=[
            pl.BlockSpec(memory_space=pltpu.SMEM),
            pl.BlockSpec((blk, d), lambda i: (i, 0)),
            pl.BlockSpec((NC, blk, d), lambda i: (0, i, 0)),
        ],
        out_specs=pl.BlockSpec((blk, d), lambda i: (i, 0)),
        out_shape=jax.ShapeDtypeStruct((n, d), jnp.float32),
    )(eps, x, partial)
    return out


# speedup vs baseline: 1.0017x; 1.0017x over previous
import functools

import jax
import jax.numpy as jnp
from jax import lax
from jax.experimental import pallas as pl
from jax.experimental.pallas import tpu as pltpu
from jax.experimental.pallas import tpu_sc as plsc

NC = 2
NS = 16
LANES = 16

CHUNK = 32
RING = 9
GLAG = 6


def _sc_partial_agg(x, edge_index, n_pad):
    n, d = x.shape
    e = edge_index.shape[1]
    e_sub = e // (NC * NS)
    n_chunks = e_sub // CHUNK
    tail = e_sub - n_chunks * CHUNK
    rows_per_sub = n_pad // NS
    dummy = n

    zeros = jnp.zeros((rows_per_sub, d), jnp.float32)

    mesh = plsc.VectorSubcoreMesh(core_axis_name="c", subcore_axis_name="s")

    @functools.partial(
        pl.kernel,
        out_type=jax.ShapeDtypeStruct((NC, n_pad, d), jnp.float32),
        mesh=mesh,
        scratch_types=[
            pltpu.VMEM((RING, CHUNK), jnp.int32),
            pltpu.VMEM((RING, CHUNK), jnp.int32),
            pltpu.VMEM((RING, CHUNK), jnp.int32),
            pltpu.VMEM((RING, CHUNK, d), jnp.float32),
            pltpu.VMEM_SHARED((n_pad, d), jnp.float32),
            pltpu.SemaphoreType.DMA((RING,)),
            pltpu.SemaphoreType.DMA((RING,)),
            pltpu.SemaphoreType.DMA((RING,)),
            pltpu.SemaphoreType.DMA,
        ],
    )
    def sc_kernel(x_hbm, ei_hbm, z_hbm, out_hbm,
                  dstb, srcb, dstm, rows, acc, isem, gsem, ssem, zsem):
        c = lax.axis_index("c")
        s = lax.axis_index("s")
        w = c * NS + s
        base = w * e_sub

        def start_idx(k, b):
            pltpu.async_copy(
                ei_hbm.at[pl.ds(base + k * CHUNK, CHUNK)], dstb.at[b],
                isem.at[b])
            pltpu.async_copy(
                ei_hbm.at[pl.ds(e + base + k * CHUNK, CHUNK)], srcb.at[b],
                isem.at[b])

        def wait_idx(b):
            pltpu.make_async_copy(
                ei_hbm.at[pl.ds(0, CHUNK)], dstb.at[b], isem.at[b]).wait()
            pltpu.make_async_copy(
                ei_hbm.at[pl.ds(0, CHUNK)], srcb.at[b], isem.at[b]).wait()

        def start_gather(b):
            pltpu.async_copy(
                x_hbm.at[srcb.at[b]], rows.at[b], gsem.at[b])

        def wait_gather(b):
            pltpu.make_async_copy(
                x_hbm.at[srcb.at[b]], rows.at[b], gsem.at[b]).wait()

        def start_scatter(b):
            pltpu.async_copy(
                rows.at[b], acc.at[dstm.at[b]], ssem.at[b], add=True)

        def wait_scatter(b):
            pltpu.make_async_copy(
                rows.at[b], acc.at[dstm.at[0]], ssem.at[b]).wait()

        def mask(b):
            for i in range(0, CHUNK, LANES):
                dsl = dstb[b, pl.ds(i, LANES)]
                ssl = srcb[b, pl.ds(i, LANES)]
                dstm[b, pl.ds(i, LANES)] = jnp.where(dsl != ssl, dsl, dummy)

        q = (rows_per_sub // 32) * 8
        zq = [q, q, q, rows_per_sub - 3 * q]
        zoff = [0, q, 2 * q, 3 * q]
        for j in range(4):
            pltpu.async_copy(
                z_hbm.at[pl.ds(zoff[j], zq[j])],
                acc.at[pl.ds(s * rows_per_sub + zoff[j], zq[j])], zsem)
        for j in range(2):
            start_idx(j, j)
        for j in range(4):
            pltpu.make_async_copy(
                z_hbm.at[pl.ds(zoff[j], zq[j])],
                acc.at[pl.ds(s * rows_per_sub + zoff[j], zq[j])], zsem).wait()
        plsc.subcore_barrier()

        total_steps = n_chunks + RING
        loop_hi = ((total_steps + RING - 1) // RING) * RING

        @pl.loop(0, loop_hi, step=RING)
        def _(k0):
            for j in range(RING):
                k = k0 + j
                b = j
                bg = (j - GLAG) % RING
                bi = (j + 2) % RING

                @pl.when(jnp.logical_and(k >= RING, k < n_chunks + RING))
                def _():
                    wait_scatter(b)

                @pl.when(k < n_chunks)
                def _():
                    wait_idx(b)
                    start_gather(b)
                    mask(b)

                @pl.when(jnp.logical_and(k >= GLAG, k < n_chunks + GLAG))
                def _():
                    wait_gather(bg)
                    start_scatter(bg)

                @pl.when(k + 2 < n_chunks)
                def _():
                    start_idx(k + 2, bi)

        if tail > 0:
            pltpu.sync_copy(
                ei_hbm.at[pl.ds(base + n_chunks * CHUNK, tail)],
                dstb.at[0, pl.ds(0, tail)])
            pltpu.sync_copy(
                ei_hbm.at[pl.ds(e + base + n_chunks * CHUNK, tail)],
                srcb.at[0, pl.ds(0, tail)])
            for i in range(0, CHUNK, LANES):
                if i + LANES <= tail:
                    dsl = dstb[0, pl.ds(i, LANES)]
                    ssl = srcb[0, pl.ds(i, LANES)]
                    dstm[0, pl.ds(i, LANES)] = jnp.where(dsl != ssl, dsl, dummy)
                else:
                    dstm[0, pl.ds(i, LANES)] = jnp.full((LANES,), dummy, jnp.int32)
                    srcb[0, pl.ds(i, LANES)] = jnp.zeros((LANES,), jnp.int32)
            start_gather(0)
            wait_gather(0)
            start_scatter(0)
            wait_scatter(0)

        plsc.subcore_barrier()

        for j in range(4):
            pltpu.async_copy(
                acc.at[pl.ds(s * rows_per_sub + zoff[j], zq[j])],
                out_hbm.at[c, pl.ds(s * rows_per_sub + zoff[j], zq[j])], zsem)
        for j in range(4):
            pltpu.make_async_copy(
                acc.at[pl.ds(s * rows_per_sub + zoff[j], zq[j])],
                out_hbm.at[c, pl.ds(s * rows_per_sub + zoff[j], zq[j])], zsem).wait()

    return sc_kernel(x, edge_index.reshape(-1), zeros)


def _tc_combine_body(eps_ref, x_ref, p_ref, o_ref):
    scale = 1.0 + eps_ref[0]
    o_ref[...] = scale * x_ref[...] + p_ref[0] + p_ref[1]


def kernel(x, edge_index, eps):
    n, d = x.shape
    n_pad = 10112
    partial = _sc_partial_agg(x, edge_index, n_pad)

    blk = 1000
    grid = (n // blk,)
    out = pl.pallas_call(
        _tc_combine_body,
        grid=grid,
        in_specs=[
            pl.BlockSpec(memory_space=pltpu.SMEM),
            pl.BlockSpec((blk, d), lambda i: (i, 0)),
            pl.BlockSpec((NC, blk, d), lambda i: (0, i, 0)),
        ],
        out_specs=pl.BlockSpec((blk, d), lambda i: (i, 0)),
        out_shape=jax.ShapeDtypeStruct((n, d), jnp.float32),
    )(eps, x, partial)
    return out


# speedup vs baseline: 1.0238x; 1.0220x over previous
import functools

import jax
import jax.numpy as jnp
from jax import lax
from jax.experimental import pallas as pl
from jax.experimental.pallas import tpu as pltpu
from jax.experimental.pallas import tpu_sc as plsc

NC = 2
NS = 16
LANES = 16

CHUNK = 32
RING = 8
GLAG = 6


def _sc_partial_agg(x, edge_index, n_pad):
    n, d = x.shape
    e = edge_index.shape[1]
    e_sub = e // (NC * NS)
    n_chunks = e_sub // CHUNK
    tail = e_sub - n_chunks * CHUNK
    rows_per_sub = n_pad // NS
    dummy = n

    zeros = jnp.zeros((rows_per_sub, d), jnp.float32)

    mesh = plsc.VectorSubcoreMesh(core_axis_name="c", subcore_axis_name="s")

    @functools.partial(
        pl.kernel,
        out_type=jax.ShapeDtypeStruct((NC, n_pad, d), jnp.float32),
        mesh=mesh,
        scratch_types=[
            pltpu.VMEM((RING, CHUNK), jnp.int32),
            pltpu.VMEM((RING, CHUNK), jnp.int32),
            pltpu.VMEM((RING, CHUNK), jnp.int32),
            pltpu.VMEM((RING, CHUNK, d), jnp.float32),
            pltpu.VMEM_SHARED((n_pad, d), jnp.float32),
            pltpu.SemaphoreType.DMA((RING,)),
            pltpu.SemaphoreType.DMA((RING,)),
            pltpu.SemaphoreType.DMA((RING,)),
            pltpu.SemaphoreType.DMA,
        ],
    )
    def sc_kernel(x_hbm, ei_hbm, z_hbm, out_hbm,
                  dstb, srcb, dstm, rows, acc, isem, gsem, ssem, zsem):
        c = lax.axis_index("c")
        s = lax.axis_index("s")
        w = c * NS + s
        base = w * e_sub

        def start_idx(k, b):
            pltpu.async_copy(
                ei_hbm.at[pl.ds(base + k * CHUNK, CHUNK)], dstb.at[b],
                isem.at[b])
            pltpu.async_copy(
                ei_hbm.at[pl.ds(e + base + k * CHUNK, CHUNK)], srcb.at[b],
                isem.at[b])

        def wait_idx(b):
            pltpu.make_async_copy(
                ei_hbm.at[pl.ds(0, CHUNK)], dstb.at[b], isem.at[b]).wait()
            pltpu.make_async_copy(
                ei_hbm.at[pl.ds(0, CHUNK)], srcb.at[b], isem.at[b]).wait()

        def start_gather(b):
            pltpu.async_copy(
                x_hbm.at[srcb.at[b]], rows.at[b], gsem.at[b])

        def wait_gather(b):
            pltpu.make_async_copy(
                x_hbm.at[srcb.at[b]], rows.at[b], gsem.at[b]).wait()

        def start_scatter(b):
            pass

        def wait_scatter(b):
            pass

        def mask(b):
            for i in range(0, CHUNK, LANES):
                dsl = dstb[b, pl.ds(i, LANES)]
                ssl = srcb[b, pl.ds(i, LANES)]
                dstm[b, pl.ds(i, LANES)] = jnp.where(dsl != ssl, dsl, dummy)

        q = (rows_per_sub // 32) * 8
        zq = [q, q, q, rows_per_sub - 3 * q]
        zoff = [0, q, 2 * q, 3 * q]
        for j in range(4):
            pltpu.async_copy(
                z_hbm.at[pl.ds(zoff[j], zq[j])],
                acc.at[pl.ds(s * rows_per_sub + zoff[j], zq[j])], zsem)
        for j in range(2):
            start_idx(j, j)
        for j in range(4):
            pltpu.make_async_copy(
                z_hbm.at[pl.ds(zoff[j], zq[j])],
                acc.at[pl.ds(s * rows_per_sub + zoff[j], zq[j])], zsem).wait()
        plsc.subcore_barrier()

        total_steps = n_chunks + RING
        loop_hi = ((total_steps + RING - 1) // RING) * RING

        @pl.loop(0, loop_hi, step=RING)
        def _(k0):
            for j in range(RING):
                k = k0 + j
                b = j
                bg = (j - GLAG) % RING
                bi = (j + 2) % RING

                @pl.when(jnp.logical_and(k >= RING, k < n_chunks + RING))
                def _():
                    wait_scatter(b)

                @pl.when(k < n_chunks)
                def _():
                    wait_idx(b)
                    start_gather(b)
                    mask(b)

                @pl.when(jnp.logical_and(k >= GLAG, k < n_chunks + GLAG))
                def _():
                    wait_gather(bg)
                    start_scatter(bg)

                @pl.when(k + 2 < n_chunks)
                def _():
                    start_idx(k + 2, bi)

        if tail > 0:
            pltpu.sync_copy(
                ei_hbm.at[pl.ds(base + n_chunks * CHUNK, tail)],
                dstb.at[0, pl.ds(0, tail)])
            pltpu.sync_copy(
                ei_hbm.at[pl.ds(e + base + n_chunks * CHUNK, tail)],
                srcb.at[0, pl.ds(0, tail)])
            for i in range(0, CHUNK, LANES):
                if i + LANES <= tail:
                    dsl = dstb[0, pl.ds(i, LANES)]
                    ssl = srcb[0, pl.ds(i, LANES)]
                    dstm[0, pl.ds(i, LANES)] = jnp.where(dsl != ssl, dsl, dummy)
                else:
                    dstm[0, pl.ds(i, LANES)] = jnp.full((LANES,), dummy, jnp.int32)
                    srcb[0, pl.ds(i, LANES)] = jnp.zeros((LANES,), jnp.int32)
            start_gather(0)
            wait_gather(0)
            start_scatter(0)
            wait_scatter(0)

        plsc.subcore_barrier()

        for j in range(4):
            pltpu.async_copy(
                acc.at[pl.ds(s * rows_per_sub + zoff[j], zq[j])],
                out_hbm.at[c, pl.ds(s * rows_per_sub + zoff[j], zq[j])], zsem)
        for j in range(4):
            pltpu.make_async_copy(
                acc.at[pl.ds(s * rows_per_sub + zoff[j], zq[j])],
                out_hbm.at[c, pl.ds(s * rows_per_sub + zoff[j], zq[j])], zsem).wait()

    return sc_kernel(x, edge_index.reshape(-1), zeros)


def _tc_combine_body(eps_ref, x_ref, p_ref, o_ref):
    scale = 1.0 + eps_ref[0]
    o_ref[...] = scale * x_ref[...] + p_ref[0] + p_ref[1]


def kernel(x, edge_index, eps):
    n, d = x.shape
    n_pad = 10112
    partial = _sc_partial_agg(x, edge_index, n_pad)

    blk = 1000
    grid = (n // blk,)
    out = pl.pallas_call(
        _tc_combine_body,
        grid=grid,
        in_specs=[
            pl.BlockSpec(memory_space=pltpu.SMEM),
            pl.BlockSpec((blk, d), lambda i: (i, 0)),
            pl.BlockSpec((NC, blk, d), lambda i: (0, i, 0)),
        ],
        out_specs=pl.BlockSpec((blk, d), lambda i: (i, 0)),
        out_shape=jax.ShapeDtypeStruct((n, d), jnp.float32),
    )(eps, x, partial)
    return out


# speedup vs baseline: 1.0309x; 1.0070x over previous
import functools

import jax
import jax.numpy as jnp
from jax import lax
from jax.experimental import pallas as pl
from jax.experimental.pallas import tpu as pltpu
from jax.experimental.pallas import tpu_sc as plsc

NC = 2
NS = 16
LANES = 16

CHUNK = 32
RING = 8
GLAG = 6
IBLK = 24


def _sc_partial_agg(x, edge_index, n_pad):
    n, d = x.shape
    e = edge_index.shape[1]
    e_sub = e // (NC * NS)
    n_chunks = e_sub // CHUNK
    tail = e_sub - n_chunks * CHUNK
    n_blocks = n_chunks // IBLK
    assert n_blocks * IBLK == n_chunks and IBLK % RING == 0
    iw = IBLK * CHUNK
    rows_per_sub = n_pad // NS
    dummy = n

    zeros = jnp.zeros((rows_per_sub, d), jnp.float32)

    mesh = plsc.VectorSubcoreMesh(core_axis_name="c", subcore_axis_name="s")

    @functools.partial(
        pl.kernel,
        out_type=jax.ShapeDtypeStruct((NC, n_pad, d), jnp.float32),
        mesh=mesh,
        scratch_types=[
            pltpu.VMEM((2 * iw,), jnp.int32),
            pltpu.VMEM((2 * iw,), jnp.int32),
            pltpu.VMEM((RING, CHUNK), jnp.int32),
            pltpu.VMEM((RING, CHUNK, d), jnp.float32),
            pltpu.VMEM_SHARED((n_pad, d), jnp.float32),
            pltpu.SemaphoreType.DMA((2,)),
            pltpu.SemaphoreType.DMA((RING,)),
            pltpu.SemaphoreType.DMA((RING,)),
            pltpu.SemaphoreType.DMA,
        ],
    )
    def sc_kernel(x_hbm, ei_hbm, z_hbm, out_hbm,
                  dstbig, srcbig, dstm, rows, acc, isem, gsem, ssem, zsem):
        c = lax.axis_index("c")
        s = lax.axis_index("s")
        w = c * NS + s
        base = w * e_sub

        def start_idxblk(blk, ib):
            pltpu.async_copy(
                ei_hbm.at[pl.ds(base + blk * iw, iw)],
                dstbig.at[pl.ds(ib * iw, iw)], isem.at[ib])
            pltpu.async_copy(
                ei_hbm.at[pl.ds(e + base + blk * iw, iw)],
                srcbig.at[pl.ds(ib * iw, iw)], isem.at[ib])

        def wait_idxblk(ib):
            pltpu.make_async_copy(
                ei_hbm.at[pl.ds(0, iw)],
                dstbig.at[pl.ds(ib * iw, iw)], isem.at[ib]).wait()
            pltpu.make_async_copy(
                ei_hbm.at[pl.ds(0, iw)],
                srcbig.at[pl.ds(ib * iw, iw)], isem.at[ib]).wait()

        def start_gather(off, b):
            pltpu.async_copy(
                x_hbm.at[srcbig.at[pl.ds(off, CHUNK)]], rows.at[b],
                gsem.at[b])

        def wait_gather(off, b):
            pltpu.make_async_copy(
                x_hbm.at[srcbig.at[pl.ds(off, CHUNK)]], rows.at[b],
                gsem.at[b]).wait()

        def start_scatter(b):
            pltpu.async_copy(
                rows.at[b], acc.at[dstm.at[b]], ssem.at[b], add=True)

        def wait_scatter(b):
            pltpu.make_async_copy(
                rows.at[b], acc.at[dstm.at[0]], ssem.at[b]).wait()

        def mask(off, b):
            for i in range(0, CHUNK, LANES):
                dsl = dstbig[pl.ds(off + i, LANES)]
                ssl = srcbig[pl.ds(off + i, LANES)]
                dstm[b, pl.ds(i, LANES)] = jnp.where(dsl != ssl, dsl, dummy)

        q = (rows_per_sub // 32) * 8
        zq = [q, q, q, rows_per_sub - 3 * q]
        zoff = [0, q, 2 * q, 3 * q]
        for j in range(4):
            pltpu.async_copy(
                z_hbm.at[pl.ds(zoff[j], zq[j])],
                acc.at[pl.ds(s * rows_per_sub + zoff[j], zq[j])], zsem)
        start_idxblk(0, 0)
        for j in range(4):
            pltpu.make_async_copy(
                z_hbm.at[pl.ds(zoff[j], zq[j])],
                acc.at[pl.ds(s * rows_per_sub + zoff[j], zq[j])], zsem).wait()
        plsc.subcore_barrier()

        @pl.loop(0, n_blocks + 1, step=2)
        def _(blk0):
            for bb in range(2):
                blk = blk0 + bb
                ib = bb
                ibn = 1 - bb

                @pl.when(blk < n_blocks)
                def _():
                    wait_idxblk(ib)

                for t in range(IBLK):
                    k = blk * IBLK + t
                    b = t % RING
                    bg = (t - GLAG) % RING
                    off = ib * iw + t * CHUNK

                    @pl.when(jnp.logical_and(k >= RING, k < n_chunks + RING))
                    def _():
                        wait_scatter(b)

                    @pl.when(k < n_chunks)
                    def _():
                        start_gather(off, b)
                        mask(off, b)

                    @pl.when(jnp.logical_and(k >= GLAG, k < n_chunks + GLAG))
                    def _():
                        goff = (ib if t >= GLAG else ibn) * iw \
                            + ((t - GLAG) % IBLK) * CHUNK
                        wait_gather(goff, bg)
                        start_scatter(bg)

                    if t == 8:
                        @pl.when(blk + 1 < n_blocks)
                        def _():
                            start_idxblk(blk + 1, ibn)

        if tail > 0:
            pltpu.sync_copy(
                ei_hbm.at[pl.ds(base + n_chunks * CHUNK, tail)],
                dstbig.at[pl.ds(0, tail)])
            pltpu.sync_copy(
                ei_hbm.at[pl.ds(e + base + n_chunks * CHUNK, tail)],
                srcbig.at[pl.ds(0, tail)])
            for i in range(0, CHUNK, LANES):
                if i + LANES <= tail:
                    dsl = dstbig[pl.ds(i, LANES)]
                    ssl = srcbig[pl.ds(i, LANES)]
                    dstm[0, pl.ds(i, LANES)] = jnp.where(dsl != ssl, dsl, dummy)
                else:
                    dstm[0, pl.ds(i, LANES)] = jnp.full((LANES,), dummy, jnp.int32)
                    srcbig[pl.ds(i, LANES)] = jnp.zeros((LANES,), jnp.int32)
            start_gather(0, 0)
            wait_gather(0, 0)
            start_scatter(0)
            wait_scatter(0)

        plsc.subcore_barrier()

        for j in range(4):
            pltpu.async_copy(
                acc.at[pl.ds(s * rows_per_sub + zoff[j], zq[j])],
                out_hbm.at[c, pl.ds(s * rows_per_sub + zoff[j], zq[j])], zsem)
        for j in range(4):
            pltpu.make_async_copy(
                acc.at[pl.ds(s * rows_per_sub + zoff[j], zq[j])],
                out_hbm.at[c, pl.ds(s * rows_per_sub + zoff[j], zq[j])], zsem).wait()

    return sc_kernel(x, edge_index.reshape(-1), zeros)


def _tc_combine_body(eps_ref, x_ref, p_ref, o_ref):
    scale = 1.0 + eps_ref[0]
    o_ref[...] = scale * x_ref[...] + p_ref[0] + p_ref[1]


def kernel(x, edge_index, eps):
    n, d = x.shape
    n_pad = 10112
    partial = _sc_partial_agg(x, edge_index, n_pad)

    blk = 1000
    grid = (n // blk,)
    out = pl.pallas_call(
        _tc_combine_body,
        grid=grid,
        in_specs=[
            pl.BlockSpec(memory_space=pltpu.SMEM),
            pl.BlockSpec((blk, d), lambda i: (i, 0)),
            pl.BlockSpec((NC, blk, d), lambda i: (0, i, 0)),
        ],
        out_specs=pl.BlockSpec((blk, d), lambda i: (i, 0)),
        out_shape=jax.ShapeDtypeStruct((n, d), jnp.float32),
    )(eps, x, partial)
    return out


# speedup vs baseline: 1.0342x; 1.0032x over previous
import functools

import jax
import jax.numpy as jnp
from jax import lax
from jax.experimental import pallas as pl
from jax.experimental.pallas import tpu as pltpu
from jax.experimental.pallas import tpu_sc as plsc

NC = 2
NS = 16
LANES = 16

CHUNK = 32
RING = 8
GLAG = 6
IBLK = 24


def _sc_partial_agg(x, edge_index, n_pad):
    n, d = x.shape
    e = edge_index.shape[1]
    e_sub = e // (NC * NS)
    n_chunks = e_sub // CHUNK
    tail = e_sub - n_chunks * CHUNK
    n_blocks = n_chunks // IBLK
    assert n_blocks * IBLK == n_chunks and IBLK % RING == 0
    iw = IBLK * CHUNK
    rows_per_sub = n_pad // NS
    dummy = n

    zeros = jnp.zeros((rows_per_sub, d), jnp.float32)

    mesh = plsc.VectorSubcoreMesh(core_axis_name="c", subcore_axis_name="s")

    @functools.partial(
        pl.kernel,
        out_type=jax.ShapeDtypeStruct((NC, n_pad, d), jnp.float32),
        mesh=mesh,
        scratch_types=[
            pltpu.VMEM((2 * iw,), jnp.int32),
            pltpu.VMEM((2 * iw,), jnp.int32),
            pltpu.VMEM((RING, CHUNK), jnp.int32),
            pltpu.VMEM((RING, CHUNK, d), jnp.float32),
            pltpu.VMEM_SHARED((n_pad, d), jnp.float32),
            pltpu.SemaphoreType.DMA((2,)),
            pltpu.SemaphoreType.DMA((RING,)),
            pltpu.SemaphoreType.DMA((RING,)),
            pltpu.SemaphoreType.DMA,
        ],
    )
    def sc_kernel(x_hbm, ei_hbm, z_hbm, out_hbm,
                  dstbig, srcbig, dstm, rows, acc, isem, gsem, ssem, zsem):
        c = lax.axis_index("c")
        s = lax.axis_index("s")
        w = c * NS + s
        base = w * e_sub

        def start_idxblk(blk, ib):
            pltpu.async_copy(
                ei_hbm.at[pl.ds(base + blk * iw, iw)],
                dstbig.at[pl.ds(ib * iw, iw)], isem.at[ib])
            pltpu.async_copy(
                ei_hbm.at[pl.ds(e + base + blk * iw, iw)],
                srcbig.at[pl.ds(ib * iw, iw)], isem.at[ib])

        def wait_idxblk(ib):
            pltpu.make_async_copy(
                ei_hbm.at[pl.ds(0, iw)],
                dstbig.at[pl.ds(ib * iw, iw)], isem.at[ib]).wait()
            pltpu.make_async_copy(
                ei_hbm.at[pl.ds(0, iw)],
                srcbig.at[pl.ds(ib * iw, iw)], isem.at[ib]).wait()

        def start_gather(off, b):
            pltpu.async_copy(
                x_hbm.at[srcbig.at[pl.ds(off, CHUNK)]], rows.at[b],
                gsem.at[b])

        def wait_gather(off, b):
            pltpu.make_async_copy(
                x_hbm.at[srcbig.at[pl.ds(off, CHUNK)]], rows.at[b],
                gsem.at[b]).wait()

        def start_scatter(b):
            pltpu.async_copy(
                rows.at[b], acc.at[dstm.at[b]], ssem.at[b], add=True)

        def wait_scatter(b):
            pltpu.make_async_copy(
                rows.at[b], acc.at[dstm.at[0]], ssem.at[b]).wait()

        def mask(off, b):
            for i in range(0, CHUNK, LANES):
                dsl = dstbig[pl.ds(off + i, LANES)]
                ssl = srcbig[pl.ds(off + i, LANES)]
                dstm[b, pl.ds(i, LANES)] = jnp.where(dsl != ssl, dsl, dummy)

        q = (rows_per_sub // 32) * 8
        zq = [q, q, q, rows_per_sub - 3 * q]
        zoff = [0, q, 2 * q, 3 * q]
        start_idxblk(0, 0)
        for j in range(4):
            pltpu.async_copy(
                z_hbm.at[pl.ds(zoff[j], zq[j])],
                acc.at[pl.ds(s * rows_per_sub + zoff[j], zq[j])], zsem)
        wait_idxblk(0)
        for t in range(GLAG):
            start_gather(t * CHUNK, t)
            mask(t * CHUNK, t)
        for j in range(4):
            pltpu.make_async_copy(
                z_hbm.at[pl.ds(zoff[j], zq[j])],
                acc.at[pl.ds(s * rows_per_sub + zoff[j], zq[j])], zsem).wait()
        plsc.subcore_barrier()

        @pl.loop(0, n_blocks + 1, step=2)
        def _(blk0):
            for bb in range(2):
                blk = blk0 + bb
                ib = bb
                ibn = 1 - bb

                @pl.when(jnp.logical_and(blk >= 1, blk < n_blocks))
                def _():
                    wait_idxblk(ib)

                for t in range(IBLK):
                    k = blk * IBLK + t
                    b = t % RING
                    bg = (t - GLAG) % RING
                    off = ib * iw + t * CHUNK

                    @pl.when(jnp.logical_and(k >= RING, k < n_chunks + RING))
                    def _():
                        wait_scatter(b)

                    @pl.when(jnp.logical_and(k >= GLAG, k < n_chunks))
                    def _():
                        start_gather(off, b)
                        mask(off, b)

                    @pl.when(jnp.logical_and(k >= GLAG, k < n_chunks + GLAG))
                    def _():
                        goff = (ib if t >= GLAG else ibn) * iw \
                            + ((t - GLAG) % IBLK) * CHUNK
                        wait_gather(goff, bg)
                        start_scatter(bg)

                    if t == 8:
                        @pl.when(blk + 1 < n_blocks)
                        def _():
                            start_idxblk(blk + 1, ibn)

        if tail > 0:
            pltpu.sync_copy(
                ei_hbm.at[pl.ds(base + n_chunks * CHUNK, tail)],
                dstbig.at[pl.ds(0, tail)])
            pltpu.sync_copy(
                ei_hbm.at[pl.ds(e + base + n_chunks * CHUNK, tail)],
                srcbig.at[pl.ds(0, tail)])
            for i in range(0, CHUNK, LANES):
                if i + LANES <= tail:
                    dsl = dstbig[pl.ds(i, LANES)]
                    ssl = srcbig[pl.ds(i, LANES)]
                    dstm[0, pl.ds(i, LANES)] = jnp.where(dsl != ssl, dsl, dummy)
                else:
                    dstm[0, pl.ds(i, LANES)] = jnp.full((LANES,), dummy, jnp.int32)
                    srcbig[pl.ds(i, LANES)] = jnp.zeros((LANES,), jnp.int32)
            start_gather(0, 0)
            wait_gather(0, 0)
            start_scatter(0)
            wait_scatter(0)

        plsc.subcore_barrier()

        for j in range(4):
            pltpu.async_copy(
                acc.at[pl.ds(s * rows_per_sub + zoff[j], zq[j])],
                out_hbm.at[c, pl.ds(s * rows_per_sub + zoff[j], zq[j])], zsem)
        for j in range(4):
            pltpu.make_async_copy(
                acc.at[pl.ds(s * rows_per_sub + zoff[j], zq[j])],
                out_hbm.at[c, pl.ds(s * rows_per_sub + zoff[j], zq[j])], zsem).wait()

    return sc_kernel(x, edge_index.reshape(-1), zeros)


def _tc_combine_body(eps_ref, x_ref, p_ref, o_ref):
    scale = 1.0 + eps_ref[0]
    o_ref[...] = scale * x_ref[...] + p_ref[0] + p_ref[1]


def kernel(x, edge_index, eps):
    n, d = x.shape
    n_pad = 10112
    partial = _sc_partial_agg(x, edge_index, n_pad)

    blk = 1000
    grid = (n // blk,)
    out = pl.pallas_call(
        _tc_combine_body,
        grid=grid,
        in_specs=[
            pl.BlockSpec(memory_space=pltpu.SMEM),
            pl.BlockSpec((blk, d), lambda i: (i, 0)),
            pl.BlockSpec((NC, blk, d), lambda i: (0, i, 0)),
        ],
        out_specs=pl.BlockSpec((blk, d), lambda i: (i, 0)),
        out_shape=jax.ShapeDtypeStruct((n, d), jnp.float32),
    )(eps, x, partial)
    return out
